# baseline (speedup 1.0000x reference)
import functools

import jax
import jax.numpy as jnp
from jax.experimental import pallas as pl

N = 4096
D_IN = 128
D_HID = 64
BR = 256


def _dot1(a, b):
    return jax.lax.dot_general(
        a.astype(jnp.bfloat16),
        b.astype(jnp.bfloat16),
        (((1,), (0,)), ((), ())),
        preferred_element_type=jnp.float32,
    )


def _prep_kernel(x_ref, w_ref, v_ref, xw_ref, s_ref):
    xw = _dot1(x_ref[...], w_ref[...])
    xw_ref[...] = xw
    f1 = _dot1(xw, v_ref[0])
    f2 = _dot1(xw, v_ref[1])
    s_ref[...] = jax.nn.sigmoid(f1 + f2)


def _half_softmax(p, s):
    vals = p * s
    e = jnp.where(vals != 0, jnp.exp(vals), 0.0)
    tot = jnp.sum(e, axis=-1, keepdims=True)
    c = jnp.where(tot > 0, 0.5 / tot, 0.0)
    return e * c


def _pass_a_kernel(local_ref, long_ref, s_ref, xw_ref, att_ref, h_ref):
    s = s_ref[...]
    att = _half_softmax(long_ref[...], s) + _half_softmax(local_ref[...], s)
    att16 = att.astype(jnp.bfloat16)
    att_ref[...] = att16
    h_ref[...] = jax.lax.dot_general(
        att16,
        xw_ref[...].astype(jnp.bfloat16),
        (((1,), (0,)), ((), ())),
        preferred_element_type=jnp.float32,
    )


def _pass_b_kernel(att_ref, h_ref, wt_ref, out_ref):
    h = h_ref[...]
    mu = jnp.mean(h, axis=0, keepdims=True)
    var = jnp.mean((h - mu) ** 2, axis=0, keepdims=True)
    hn = (h - mu) / jnp.sqrt(var + 1e-6)
    dz = _dot1(att_ref[...], hn)
    dz = _dot1(dz, wt_ref[...])
    out_ref[...] = jnp.where(dz > 0, dz, jnp.exp(jnp.minimum(dz, 0.0)) - 1.0)


@functools.partial(jax.jit, static_argnames=())
def kernel(local_patten, long_range_patten, X, W, V):
    xw, s = pl.pallas_call(
        _prep_kernel,
        out_shape=(
            jax.ShapeDtypeStruct((N, D_HID), jnp.float32),
            jax.ShapeDtypeStruct((N, 1), jnp.float32),
        ),
    )(X, W, V)

    grid = (N // BR,)
    att, h = pl.pallas_call(
        _pass_a_kernel,
        grid=grid,
        in_specs=[
            pl.BlockSpec((BR, N), lambda i: (i, 0)),
            pl.BlockSpec((BR, N), lambda i: (i, 0)),
            pl.BlockSpec((BR, 1), lambda i: (i, 0)),
            pl.BlockSpec((N, D_HID), lambda i: (0, 0)),
        ],
        out_specs=(
            pl.BlockSpec((BR, N), lambda i: (i, 0)),
            pl.BlockSpec((BR, D_HID), lambda i: (i, 0)),
        ),
        out_shape=(
            jax.ShapeDtypeStruct((N, N), jnp.bfloat16),
            jax.ShapeDtypeStruct((N, D_HID), jnp.float32),
        ),
    )(local_patten, long_range_patten, s, xw)

    out = pl.pallas_call(
        _pass_b_kernel,
        grid=grid,
        in_specs=[
            pl.BlockSpec((BR, N), lambda i: (i, 0)),
            pl.BlockSpec((N, D_HID), lambda i: (0, 0)),
            pl.BlockSpec((D_HID, D_IN), lambda i: (0, 0)),
        ],
        out_specs=pl.BlockSpec((BR, D_IN), lambda i: (i, 0)),
        out_shape=jax.ShapeDtypeStruct((N, D_IN), jnp.float32),
    )(att, h, W.T)

    return out


# speedup vs baseline: 1.0803x; 1.0803x over previous
import functools

import jax
import jax.numpy as jnp
from jax.experimental import pallas as pl

N = 4096
D_IN = 128
D_HID = 64
BR = 256


def _dot1(a, b):
    return jax.lax.dot_general(
        a.astype(jnp.bfloat16),
        b.astype(jnp.bfloat16),
        (((1,), (0,)), ((), ())),
        preferred_element_type=jnp.float32,
    )


def _prep_kernel(x_ref, w_ref, v_ref, xw_ref, s_ref):
    xw = _dot1(x_ref[...], w_ref[...])
    xw_ref[...] = xw
    f1 = _dot1(xw, v_ref[0])
    f2 = _dot1(xw, v_ref[1])
    s_ref[...] = jax.nn.sigmoid(f1 + f2)


def _half_softmax(p, s):
    vals = p * s
    e = jnp.where(vals != 0, jnp.exp(vals), 0.0)
    tot = jnp.sum(e, axis=-1, keepdims=True)
    c = jnp.where(tot > 0, 0.5 / tot, 0.0)
    return e * c


def _pass_a_kernel(local_ref, long_ref, s_ref, xw_ref, att_ref, h_ref):
    s = s_ref[...]
    att = _half_softmax(long_ref[...], s) + _half_softmax(local_ref[...], s)
    att16 = att.astype(jnp.bfloat16)
    att_ref[...] = att16
    h_ref[...] = jax.lax.dot_general(
        att16,
        xw_ref[...].astype(jnp.bfloat16),
        (((1,), (0,)), ((), ())),
        preferred_element_type=jnp.float32,
    )


def _bn_kernel(h_ref, hn_ref):
    h = h_ref[...]
    mu = jnp.mean(h, axis=0, keepdims=True)
    var = jnp.mean((h - mu) ** 2, axis=0, keepdims=True)
    hn_ref[...] = (h - mu) / jnp.sqrt(var + 1e-6)


def _pass_b_kernel(att_ref, hn_ref, wt_ref, out_ref):
    dz = _dot1(att_ref[...], hn_ref[...])
    dz = _dot1(dz, wt_ref[...])
    out_ref[...] = jnp.where(dz > 0, dz, jnp.exp(jnp.minimum(dz, 0.0)) - 1.0)


@functools.partial(jax.jit, static_argnames=())
def kernel(local_patten, long_range_patten, X, W, V):
    xw, s = pl.pallas_call(
        _prep_kernel,
        out_shape=(
            jax.ShapeDtypeStruct((N, D_HID), jnp.float32),
            jax.ShapeDtypeStruct((N, 1), jnp.float32),
        ),
    )(X, W, V)

    grid = (N // BR,)
    att, h = pl.pallas_call(
        _pass_a_kernel,
        grid=grid,
        in_specs=[
            pl.BlockSpec((BR, N), lambda i: (i, 0)),
            pl.BlockSpec((BR, N), lambda i: (i, 0)),
            pl.BlockSpec((BR, 1), lambda i: (i, 0)),
            pl.BlockSpec((N, D_HID), lambda i: (0, 0)),
        ],
        out_specs=(
            pl.BlockSpec((BR, N), lambda i: (i, 0)),
            pl.BlockSpec((BR, D_HID), lambda i: (i, 0)),
        ),
        out_shape=(
            jax.ShapeDtypeStruct((N, N), jnp.bfloat16),
            jax.ShapeDtypeStruct((N, D_HID), jnp.float32),
        ),
    )(local_patten, long_range_patten, s, xw)

    hn = pl.pallas_call(
        _bn_kernel,
        out_shape=jax.ShapeDtypeStruct((N, D_HID), jnp.float32),
    )(h)

    out = pl.pallas_call(
        _pass_b_kernel,
        grid=grid,
        in_specs=[
            pl.BlockSpec((BR, N), lambda i: (i, 0)),
            pl.BlockSpec((N, D_HID), lambda i: (0, 0)),
            pl.BlockSpec((D_HID, D_IN), lambda i: (0, 0)),
        ],
        out_specs=pl.BlockSpec((BR, D_IN), lambda i: (i, 0)),
        out_shape=jax.ShapeDtypeStruct((N, D_IN), jnp.float32),
    )(att, hn, W.T)

    return out


# speedup vs baseline: 1.1756x; 1.0882x over previous
import functools

import jax
import jax.numpy as jnp
from jax.experimental import pallas as pl

N = 4096
D_IN = 128
D_HID = 64
BR = 512


def _dot1(a, b):
    return jax.lax.dot_general(
        a.astype(jnp.bfloat16),
        b.astype(jnp.bfloat16),
        (((1,), (0,)), ((), ())),
        preferred_element_type=jnp.float32,
    )


def _prep_kernel(x_ref, w_ref, v_ref, xw_ref, s_ref):
    xw = _dot1(x_ref[...], w_ref[...])
    xw_ref[...] = xw
    f1 = _dot1(xw, v_ref[0])
    f2 = _dot1(xw, v_ref[1])
    s_ref[...] = jax.nn.sigmoid(f1 + f2)


def _half_softmax(p, s):
    vals = p * s
    e = jnp.where(vals != 0, jnp.exp(vals), 0.0)
    tot = jnp.sum(e, axis=-1, keepdims=True)
    c = jnp.where(tot > 0, 0.5 / tot, 0.0)
    return e * c


def _pass_a_kernel(local_ref, long_ref, s_ref, xw_ref, att_ref, h_ref):
    s = s_ref[...]
    att = _half_softmax(long_ref[...], s) + _half_softmax(local_ref[...], s)
    att16 = att.astype(jnp.bfloat16)
    att_ref[...] = att16
    h_ref[...] = jax.lax.dot_general(
        att16,
        xw_ref[...].astype(jnp.bfloat16),
        (((1,), (0,)), ((), ())),
        preferred_element_type=jnp.float32,
    )


def _bn_kernel(h_ref, hn_ref):
    h = h_ref[...]
    mu = jnp.mean(h, axis=0, keepdims=True)
    var = jnp.mean((h - mu) ** 2, axis=0, keepdims=True)
    hn_ref[...] = (h - mu) / jnp.sqrt(var + 1e-6)


def _pass_b_kernel(att_ref, hn_ref, wt_ref, out_ref):
    dz = _dot1(att_ref[...], hn_ref[...])
    dz = _dot1(dz, wt_ref[...])
    out_ref[...] = jnp.where(dz > 0, dz, jnp.exp(jnp.minimum(dz, 0.0)) - 1.0)


@functools.partial(jax.jit, static_argnames=())
def kernel(local_patten, long_range_patten, X, W, V):
    xw, s = pl.pallas_call(
        _prep_kernel,
        out_shape=(
            jax.ShapeDtypeStruct((N, D_HID), jnp.float32),
            jax.ShapeDtypeStruct((N, 1), jnp.float32),
        ),
    )(X, W, V)

    grid = (N // BR,)
    att, h = pl.pallas_call(
        _pass_a_kernel,
        grid=grid,
        in_specs=[
            pl.BlockSpec((BR, N), lambda i: (i, 0)),
            pl.BlockSpec((BR, N), lambda i: (i, 0)),
            pl.BlockSpec((BR, 1), lambda i: (i, 0)),
            pl.BlockSpec((N, D_HID), lambda i: (0, 0)),
        ],
        out_specs=(
            pl.BlockSpec((BR, N), lambda i: (i, 0)),
            pl.BlockSpec((BR, D_HID), lambda i: (i, 0)),
        ),
        out_shape=(
            jax.ShapeDtypeStruct((N, N), jnp.bfloat16),
            jax.ShapeDtypeStruct((N, D_HID), jnp.float32),
        ),
    )(local_patten, long_range_patten, s, xw)

    hn = pl.pallas_call(
        _bn_kernel,
        out_shape=jax.ShapeDtypeStruct((N, D_HID), jnp.float32),
    )(h)

    out = pl.pallas_call(
        _pass_b_kernel,
        grid=grid,
        in_specs=[
            pl.BlockSpec((BR, N), lambda i: (i, 0)),
            pl.BlockSpec((N, D_HID), lambda i: (0, 0)),
            pl.BlockSpec((D_HID, D_IN), lambda i: (0, 0)),
        ],
        out_specs=pl.BlockSpec((BR, D_IN), lambda i: (i, 0)),
        out_shape=jax.ShapeDtypeStruct((N, D_IN), jnp.float32),
    )(att, hn, W.T)

    return out
